# Initial kernel scaffold; baseline (speedup 1.0000x reference)
import functools

import jax
import jax.numpy as jnp
from jax import lax
from jax.experimental import pallas as pl
from jax.experimental.pallas import tpu as pltpu
from jax.experimental.pallas import tpu_sc as plsc

N = 10000
E = 320000
D = 128
NC = 2
NS = 16
NW = NC * NS
L = 16
CH = 128
NCH = 79
EPW = NCH * CH
E_PAD = NW * EPW
N_PAD = 10112
RPT = N_PAD // NS
SUB = (128, 128, 128, 128, 120)

_MESH = dict(core_axis_name="c", subcore_axis_name="s", num_cores=NC,
             num_subcores=NS)


def _zero_rows(rows_v):
    def zr(i, _):
        for q in range(D // L):
            rows_v[i, pl.ds(q * L, L)] = jnp.zeros((L,), jnp.float32)
        return 0
    lax.fori_loop(0, CH, zr, 0)


def _sc_deg_body(src_hbm, dst_hbm, odp, idp, src_v, dst_v, ones_v, zz_v,
                 od_sp, id_sp, sem):
    cid = lax.axis_index("c")
    sid = lax.axis_index("s")
    wid = sid * NC + cid
    base = sid * RPT
    pltpu.sync_copy(src_hbm.at[wid], src_v)
    pltpu.sync_copy(dst_hbm.at[wid], dst_v)
    for t in range(CH // L):
        ones_v[pl.ds(t * L, L)] = jnp.ones((L,), jnp.float32)
    for t in range(640 // L):
        zz_v[pl.ds(t * L, L)] = jnp.zeros((L,), jnp.float32)
    pltpu.sync_copy(zz_v.at[pl.ds(0, RPT)], od_sp.at[pl.ds(base, RPT)])
    pltpu.sync_copy(zz_v.at[pl.ds(0, RPT)], id_sp.at[pl.ds(base, RPT)])
    plsc.subcore_barrier()

    def dchunk(j, _):
        pltpu.sync_copy(ones_v, od_sp.at[src_v.at[j]], add=True)
        pltpu.sync_copy(ones_v, id_sp.at[dst_v.at[j]], add=True)
        return 0
    lax.fori_loop(0, NCH, dchunk, 0)
    plsc.subcore_barrier()
    pltpu.sync_copy(od_sp.at[pl.ds(base, RPT)], odp.at[cid, pl.ds(base, RPT)])
    pltpu.sync_copy(id_sp.at[pl.ds(base, RPT)], idp.at[cid, pl.ds(base, RPT)])


def _sc_deg(srcw, dstw):
    mesh = plsc.VectorSubcoreMesh(**_MESH)
    f = pl.kernel(
        _sc_deg_body,
        out_type=(jax.ShapeDtypeStruct((NC, N_PAD), jnp.float32),
                  jax.ShapeDtypeStruct((NC, N_PAD), jnp.float32)),
        mesh=mesh,
        scratch_types=[
            pltpu.VMEM((NCH, CH), jnp.int32),
            pltpu.VMEM((NCH, CH), jnp.int32),
            pltpu.VMEM((CH,), jnp.float32),
            pltpu.VMEM((640,), jnp.float32),
            pltpu.VMEM_SHARED((N_PAD,), jnp.float32),
            pltpu.VMEM_SHARED((N_PAD,), jnp.float32),
            pltpu.SemaphoreType.DMA,
        ],
    )
    return f(srcw, dstw)


def _make_sc_agg_body(with_c):
    def body(*refs):
        if with_c:
            (x_hbm, src_hbm, dst_hbm, cw_hbm, ns_hbm, nd_hbm,
             c_out, agg_out,
             src_v, dst_v, c_v, rows_v, nd_v, norm_v, agg_sp, sem) = refs
        else:
            (x_hbm, src_hbm, dst_hbm, cw_hbm, nd_hbm,
             agg_out,
             src_v, dst_v, c_v, rows_v, nd_v, agg_sp, sem) = refs
        cid = lax.axis_index("c")
        sid = lax.axis_index("s")
        wid = sid * NC + cid
        base = sid * RPT
        pltpu.sync_copy(src_hbm.at[wid], src_v)
        pltpu.sync_copy(dst_hbm.at[wid], dst_v)
        pltpu.sync_copy(cw_hbm.at[wid], c_v)
        pltpu.sync_copy(nd_hbm.at[pl.ds(base, RPT)], nd_v)
        if with_c:
            pltpu.sync_copy(ns_hbm, norm_v)
        _zero_rows(rows_v)
        off = 0
        for sz in SUB:
            pltpu.sync_copy(rows_v.at[pl.ds(0, sz)],
                            agg_sp.at[pl.ds(base + off, sz), :])
            off += sz
        if with_c:
            def cbody(j, _):
                for t in range(CH // L):
                    idx = src_v[j, pl.ds(t * L, L)]
                    nv = plsc.load_gather(norm_v, [idx])
                    c_v[j, pl.ds(t * L, L)] = c_v[j, pl.ds(t * L, L)] * nv
                return 0
            lax.fori_loop(0, NCH, cbody, 0)
            pltpu.sync_copy(c_v, c_out.at[wid])
        plsc.subcore_barrier()

        def chunk(j, _):
            pltpu.async_copy(x_hbm.at[src_v.at[j]], rows_v, sem).wait()

            def scale(k, _):
                cv = lax.broadcast(c_v[j, k], (L,))
                for q in range(D // L):
                    rows_v[k, pl.ds(q * L, L)] = (
                        rows_v[k, pl.ds(q * L, L)] * cv)
                return 0
            lax.fori_loop(0, CH, scale, 0)
            pltpu.sync_copy(rows_v, agg_sp.at[dst_v.at[j]], add=True)
            return 0
        lax.fori_loop(0, NCH, chunk, 0)
        plsc.subcore_barrier()
        off = 0
        for sz in SUB:
            pltpu.sync_copy(agg_sp.at[pl.ds(base + off, sz), :],
                            rows_v.at[pl.ds(0, sz)])

            def nrow(r, _):
                cv = lax.broadcast(nd_v[off + r], (L,))
                for q in range(D // L):
                    rows_v[r, pl.ds(q * L, L)] = (
                        rows_v[r, pl.ds(q * L, L)] * cv)
                return 0
            lax.fori_loop(0, sz, nrow, 0)
            pltpu.sync_copy(rows_v.at[pl.ds(0, sz)],
                            agg_out.at[cid, pl.ds(base + off, sz), :])
            off += sz
    return body


def _sc_agg(with_c, *args):
    mesh = plsc.VectorSubcoreMesh(**_MESH)
    scratch = [
        pltpu.VMEM((NCH, CH), jnp.int32),
        pltpu.VMEM((NCH, CH), jnp.int32),
        pltpu.VMEM((NCH, CH), jnp.float32),
        pltpu.VMEM((CH, D), jnp.float32),
        pltpu.VMEM((RPT,), jnp.float32),
    ]
    if with_c:
        scratch.append(pltpu.VMEM((N_PAD,), jnp.float32))
    scratch += [
        pltpu.VMEM_SHARED((N_PAD, D), jnp.float32),
        pltpu.SemaphoreType.DMA,
    ]
    out_type = [jax.ShapeDtypeStruct((NC, N_PAD, D), jnp.float32)]
    if with_c:
        out_type = [jax.ShapeDtypeStruct((NW, NCH, CH), jnp.float32)] + out_type
    f = pl.kernel(
        _make_sc_agg_body(with_c),
        out_type=tuple(out_type),
        mesh=mesh,
        scratch_types=scratch,
    )
    return f(*args)


def _tc_ew_body(e_ref, l1w_ref, l1b_ref, l2w_ref, l2b_ref, o_ref):
    w_eff = jnp.sum(l1w_ref[...] * l2w_ref[...][None, :], axis=1)
    b_eff = jnp.sum(l1b_ref[...] * l2w_ref[...]) + l2b_ref[...][0]
    o_ref[...] = jnp.sum(e_ref[...] * w_eff[None, :], axis=1) + b_eff


def _tc_ew(edges_pad, l1w, l1b, l2w_flat, l2b):
    blk = 2048
    grid = E_PAD // blk
    return pl.pallas_call(
        _tc_ew_body,
        grid=(grid,),
        in_specs=[
            pl.BlockSpec((blk, 16), lambda i: (i, 0)),
            pl.BlockSpec((16, 8), lambda i: (0, 0)),
            pl.BlockSpec((8,), lambda i: (0,)),
            pl.BlockSpec((8,), lambda i: (0,)),
            pl.BlockSpec((1,), lambda i: (0,)),
        ],
        out_specs=pl.BlockSpec((blk,), lambda i: (i,)),
        out_shape=jax.ShapeDtypeStruct((E_PAD,), jnp.float32),
    )(edges_pad, l1w, l1b, l2w_flat, l2b)


def _tc_norm_body(odp_ref, idp_ref, ns_ref, nd_ref):
    od = odp_ref[0, :] + odp_ref[1, :]
    idg = idp_ref[0, :] + idp_ref[1, :]
    ns_ref[...] = lax.rsqrt(jnp.maximum(od, 1.0))
    nd_ref[...] = lax.rsqrt(jnp.maximum(idg, 1.0))


def _tc_norm(odp, idp):
    return pl.pallas_call(
        _tc_norm_body,
        out_shape=(jax.ShapeDtypeStruct((N_PAD,), jnp.float32),
                   jax.ShapeDtypeStruct((N_PAD,), jnp.float32)),
    )(odp, idp)


def _tc_layer_body(aggp_ref, w_ref, b_ref, o_ref):
    a = aggp_ref[0] + aggp_ref[1]
    h = jnp.dot(a, w_ref[...], preferred_element_type=jnp.float32,
                precision=lax.Precision.HIGHEST)
    o_ref[...] = jnp.maximum(h + b_ref[...][None, :], 0.0)


def _tc_layer(aggp, w, b):
    rb = RPT
    grid = N_PAD // rb
    return pl.pallas_call(
        _tc_layer_body,
        grid=(grid,),
        in_specs=[
            pl.BlockSpec((NC, rb, D), lambda i: (0, i, 0)),
            pl.BlockSpec((D, D), lambda i: (0, 0)),
            pl.BlockSpec((D,), lambda i: (0,)),
        ],
        out_specs=pl.BlockSpec((rb, D), lambda i: (i, 0)),
        out_shape=jax.ShapeDtypeStruct((N_PAD, D), jnp.float32),
    )(aggp, w, b)


def kernel(inputs, edge_index, edges, W1, b1, W2, b2, lin1_W, lin1_b,
           lin2_W, lin2_b):
    pad = E_PAD - E
    trash = jnp.full((pad,), N, jnp.int32)
    srcw = jnp.concatenate([edge_index[0], trash]).reshape(NW, NCH, CH)
    dstw = jnp.concatenate([edge_index[1], trash]).reshape(NW, NCH, CH)
    x_pad = jnp.pad(inputs, ((0, N_PAD - N), (0, 0)))
    edges_pad = jnp.pad(edges, ((0, pad), (0, 0)))
    l2w_flat = lin2_W.reshape(8)

    ew = _tc_ew(edges_pad, lin1_W, lin1_b, l2w_flat, lin2_b)
    eww = ew.reshape(NW, NCH, CH)
    odp, idp = _sc_deg(srcw, dstw)
    ns, nd = _tc_norm(odp, idp)
    c_out, agg1 = _sc_agg(True, x_pad, srcw, dstw, eww, ns, nd)
    h = _tc_layer(agg1, W1, b1)
    agg2 = _sc_agg(False, h, srcw, dstw, c_out, nd)
    out_full = _tc_layer(agg2, W2, b2)
    return out_full[:N]


# speedup vs baseline: 2.3100x; 2.3100x over previous
import functools

import jax
import jax.numpy as jnp
from jax import lax
from jax.experimental import pallas as pl
from jax.experimental.pallas import tpu as pltpu
from jax.experimental.pallas import tpu_sc as plsc

N = 10000
E = 320000
D = 128
NC = 2
NS = 16
NW = NC * NS
L = 16
CH = 128
NCH = 80
G = 8
NG = NCH // G
EPW = NCH * CH
E_PAD = NW * EPW
N_PAD = 10240
RPT = N_PAD // NS
SUB = (128, 128, 128, 128, 128)

_MESH = dict(core_axis_name="c", subcore_axis_name="s", num_cores=NC,
             num_subcores=NS)


def _zero_rows(rows_v):
    def zr(i, _):
        for q in range(D // L):
            rows_v[i, pl.ds(q * L, L)] = jnp.zeros((L,), jnp.float32)
        return 0
    lax.fori_loop(0, CH, zr, 0)


def _sc_deg_body(src_hbm, dst_hbm, odp, idp, src_v, dst_v, ones_v, zz_v,
                 od_sp, id_sp, sem):
    cid = lax.axis_index("c")
    sid = lax.axis_index("s")
    wid = sid * NC + cid
    base = sid * RPT
    pltpu.sync_copy(src_hbm.at[wid], src_v)
    pltpu.sync_copy(dst_hbm.at[wid], dst_v)
    for t in range(CH // L):
        ones_v[pl.ds(t * L, L)] = jnp.ones((L,), jnp.float32)
    for t in range(RPT // L):
        zz_v[pl.ds(t * L, L)] = jnp.zeros((L,), jnp.float32)
    pltpu.sync_copy(zz_v.at[pl.ds(0, RPT)], od_sp.at[pl.ds(base, RPT)])
    pltpu.sync_copy(zz_v.at[pl.ds(0, RPT)], id_sp.at[pl.ds(base, RPT)])
    plsc.subcore_barrier()

    def dchunk(j, _):
        pltpu.sync_copy(ones_v, od_sp.at[src_v.at[j]], add=True)
        pltpu.sync_copy(ones_v, id_sp.at[dst_v.at[j]], add=True)
        return 0
    lax.fori_loop(0, NCH, dchunk, 0)
    plsc.subcore_barrier()
    pltpu.sync_copy(od_sp.at[pl.ds(base, RPT)], odp.at[cid, pl.ds(base, RPT)])
    pltpu.sync_copy(id_sp.at[pl.ds(base, RPT)], idp.at[cid, pl.ds(base, RPT)])


def _sc_deg(srcw, dstw):
    mesh = plsc.VectorSubcoreMesh(**_MESH)
    f = pl.kernel(
        _sc_deg_body,
        out_type=(jax.ShapeDtypeStruct((NC, N_PAD), jnp.float32),
                  jax.ShapeDtypeStruct((NC, N_PAD), jnp.float32)),
        mesh=mesh,
        compiler_params=pltpu.CompilerParams(needs_layout_passes=False),
        scratch_types=[
            pltpu.VMEM((NCH, CH), jnp.int32),
            pltpu.VMEM((NCH, CH), jnp.int32),
            pltpu.VMEM((CH,), jnp.float32),
            pltpu.VMEM((RPT,), jnp.float32),
            pltpu.VMEM_SHARED((N_PAD,), jnp.float32),
            pltpu.VMEM_SHARED((N_PAD,), jnp.float32),
            pltpu.SemaphoreType.DMA,
        ],
    )
    return f(srcw, dstw)


def _make_sc_agg_body(with_c):
    def body(*refs):
        if with_c:
            (x_hbm, src_hbm, dst_hbm, cw_hbm, ns_hbm, nd_hbm,
             c_out, agg_out,
             src_g, dst_g, c_g, rows_v, nd_v, norm_v, agg_sp, sem) = refs
        else:
            (x_hbm, src_hbm, dst_hbm, cw_hbm, nd_hbm,
             agg_out,
             src_g, dst_g, c_g, rows_v, nd_v, agg_sp, sem) = refs
        cid = lax.axis_index("c")
        sid = lax.axis_index("s")
        wid = sid * NC + cid
        base = sid * RPT
        pltpu.sync_copy(nd_hbm.at[pl.ds(base, RPT)], nd_v)
        if with_c:
            pltpu.sync_copy(ns_hbm, norm_v)
        _zero_rows(rows_v)
        off = 0
        for sz in SUB:
            pltpu.sync_copy(rows_v.at[pl.ds(0, sz)],
                            agg_sp.at[pl.ds(base + off, sz), :])
            off += sz
        plsc.subcore_barrier()

        def group(grp, _):
            pltpu.sync_copy(src_hbm.at[wid, pl.ds(grp * G, G)], src_g)
            pltpu.sync_copy(dst_hbm.at[wid, pl.ds(grp * G, G)], dst_g)
            pltpu.sync_copy(cw_hbm.at[wid, pl.ds(grp * G, G)], c_g)
            if with_c:
                def cbody(j2, _):
                    for t in range(CH // L):
                        idx = src_g[j2, pl.ds(t * L, L)]
                        nv = plsc.load_gather(norm_v, [idx])
                        c_g[j2, pl.ds(t * L, L)] = (
                            c_g[j2, pl.ds(t * L, L)] * nv)
                    return 0
                lax.fori_loop(0, G, cbody, 0)
                pltpu.sync_copy(c_g, c_out.at[wid, pl.ds(grp * G, G)])
            for j in range(G):
                pltpu.async_copy(x_hbm.at[src_g.at[j]], rows_v, sem).wait()

                def scale(t, _, j=j):
                    cvec = c_g[j, pl.ds(t * L, L)]
                    for k in range(L):
                        cv = lax.broadcast(cvec[k], (L,))
                        row = t * L + k
                        for q in range(D // L):
                            rows_v[row, pl.ds(q * L, L)] = (
                                rows_v[row, pl.ds(q * L, L)] * cv)
                    return 0
                lax.fori_loop(0, CH // L, scale, 0)
                pltpu.sync_copy(rows_v, agg_sp.at[dst_g.at[j]], add=True)
            return 0
        lax.fori_loop(0, NG, group, 0)
        plsc.subcore_barrier()
        off = 0
        for sz in SUB:
            pltpu.sync_copy(agg_sp.at[pl.ds(base + off, sz), :],
                            rows_v.at[pl.ds(0, sz)])

            def nrow(g, _):
                nvec = nd_v[pl.ds(off + g * L, L)]
                for k in range(L):
                    cv = lax.broadcast(nvec[k], (L,))
                    row = g * L + k
                    for q in range(D // L):
                        rows_v[row, pl.ds(q * L, L)] = (
                            rows_v[row, pl.ds(q * L, L)] * cv)
                return 0
            lax.fori_loop(0, sz // L, nrow, 0)
            pltpu.sync_copy(rows_v.at[pl.ds(0, sz)],
                            agg_out.at[cid, pl.ds(base + off, sz), :])
            off += sz
    return body


def _sc_agg(with_c, *args):
    mesh = plsc.VectorSubcoreMesh(**_MESH)
    scratch = [
        pltpu.VMEM((G, CH), jnp.int32),
        pltpu.VMEM((G, CH), jnp.int32),
        pltpu.VMEM((G, CH), jnp.float32),
        pltpu.VMEM((CH, D), jnp.float32),
        pltpu.VMEM((RPT,), jnp.float32),
    ]
    if with_c:
        scratch.append(pltpu.VMEM((N_PAD,), jnp.float32))
    scratch += [
        pltpu.VMEM_SHARED((N_PAD, D), jnp.float32),
        pltpu.SemaphoreType.DMA,
    ]
    out_type = [jax.ShapeDtypeStruct((NC, N_PAD, D), jnp.float32)]
    if with_c:
        out_type = [jax.ShapeDtypeStruct((NW, NCH, CH), jnp.float32)] + out_type
    f = pl.kernel(
        _make_sc_agg_body(with_c),
        out_type=tuple(out_type),
        mesh=mesh,
        compiler_params=pltpu.CompilerParams(needs_layout_passes=False),
        scratch_types=scratch,
    )
    return f(*args)


def _tc_ew_body(e_ref, l1w_ref, l1b_ref, l2w_ref, l2b_ref, o_ref):
    w_eff = jnp.sum(l1w_ref[...] * l2w_ref[...][None, :], axis=1)
    b_eff = jnp.sum(l1b_ref[...] * l2w_ref[...]) + l2b_ref[...][0]
    o_ref[...] = jnp.sum(e_ref[...] * w_eff[None, :], axis=1) + b_eff


def _tc_ew(edges_pad, l1w, l1b, l2w_flat, l2b):
    blk = 2048
    grid = E_PAD // blk
    return pl.pallas_call(
        _tc_ew_body,
        grid=(grid,),
        in_specs=[
            pl.BlockSpec((blk, 16), lambda i: (i, 0)),
            pl.BlockSpec((16, 8), lambda i: (0, 0)),
            pl.BlockSpec((8,), lambda i: (0,)),
            pl.BlockSpec((8,), lambda i: (0,)),
            pl.BlockSpec((1,), lambda i: (0,)),
        ],
        out_specs=pl.BlockSpec((blk,), lambda i: (i,)),
        out_shape=jax.ShapeDtypeStruct((E_PAD,), jnp.float32),
    )(edges_pad, l1w, l1b, l2w_flat, l2b)


def _tc_norm_body(odp_ref, idp_ref, ns_ref, nd_ref):
    od = odp_ref[0, :] + odp_ref[1, :]
    idg = idp_ref[0, :] + idp_ref[1, :]
    ns_ref[...] = lax.rsqrt(jnp.maximum(od, 1.0))
    nd_ref[...] = lax.rsqrt(jnp.maximum(idg, 1.0))


def _tc_norm(odp, idp):
    return pl.pallas_call(
        _tc_norm_body,
        out_shape=(jax.ShapeDtypeStruct((N_PAD,), jnp.float32),
                   jax.ShapeDtypeStruct((N_PAD,), jnp.float32)),
    )(odp, idp)


def _tc_layer_body(aggp_ref, w_ref, b_ref, o_ref):
    a = aggp_ref[0] + aggp_ref[1]
    h = jnp.dot(a, w_ref[...], preferred_element_type=jnp.float32,
                precision=lax.Precision.HIGHEST)
    o_ref[...] = jnp.maximum(h + b_ref[...][None, :], 0.0)


def _tc_layer(aggp, w, b):
    rb = RPT
    grid = N_PAD // rb
    return pl.pallas_call(
        _tc_layer_body,
        grid=(grid,),
        in_specs=[
            pl.BlockSpec((NC, rb, D), lambda i: (0, i, 0)),
            pl.BlockSpec((D, D), lambda i: (0, 0)),
            pl.BlockSpec((D,), lambda i: (0,)),
        ],
        out_specs=pl.BlockSpec((rb, D), lambda i: (i, 0)),
        out_shape=jax.ShapeDtypeStruct((N_PAD, D), jnp.float32),
    )(aggp, w, b)


def kernel(inputs, edge_index, edges, W1, b1, W2, b2, lin1_W, lin1_b,
           lin2_W, lin2_b):
    pad = E_PAD - E
    trash = jnp.full((pad,), N, jnp.int32)
    srcw = jnp.concatenate([edge_index[0], trash]).reshape(NW, NCH, CH)
    dstw = jnp.concatenate([edge_index[1], trash]).reshape(NW, NCH, CH)
    x_pad = jnp.pad(inputs, ((0, N_PAD - N), (0, 0)))
    edges_pad = jnp.pad(edges, ((0, pad), (0, 0)))
    l2w_flat = lin2_W.reshape(8)

    ew = _tc_ew(edges_pad, lin1_W, lin1_b, l2w_flat, lin2_b)
    eww = ew.reshape(NW, NCH, CH)
    odp, idp = _sc_deg(srcw, dstw)
    ns, nd = _tc_norm(odp, idp)
    c_out, agg1 = _sc_agg(True, x_pad, srcw, dstw, eww, ns, nd)
    h = _tc_layer(agg1, W1, b1)
    (agg2,) = _sc_agg(False, h, srcw, dstw, c_out, nd)
    out_full = _tc_layer(agg2, W2, b2)
    return out_full[:N]


# speedup vs baseline: 2.5793x; 1.1166x over previous
import functools

import jax
import jax.numpy as jnp
from jax import lax
from jax.experimental import pallas as pl
from jax.experimental.pallas import tpu as pltpu
from jax.experimental.pallas import tpu_sc as plsc

N = 10000
E = 320000
D = 128
NC = 2
NS = 16
NW = NC * NS
L = 16
CH = 128
NCH = 80
G = 8
NG = NCH // G
EPW = NCH * CH
E_PAD = NW * EPW
N_PAD = 10240
RPT = N_PAD // NS
SUB = (128, 128, 128, 128, 128)

_MESH = dict(core_axis_name="c", subcore_axis_name="s", num_cores=NC,
             num_subcores=NS)


def _zero_rows(rows_v):
    def zr(i, _):
        for q in range(D // L):
            rows_v[i, pl.ds(q * L, L)] = jnp.zeros((L,), jnp.float32)
        return 0
    lax.fori_loop(0, CH, zr, 0)


def _sc_deg_body(src_hbm, dst_hbm, odp, idp, src_v, dst_v, ones_v, zz_v,
                 od_sp, id_sp, sem):
    cid = lax.axis_index("c")
    sid = lax.axis_index("s")
    wid = sid * NC + cid
    base = sid * RPT
    pltpu.sync_copy(src_hbm.at[wid], src_v)
    pltpu.sync_copy(dst_hbm.at[wid], dst_v)
    for t in range(CH // L):
        ones_v[pl.ds(t * L, L)] = jnp.ones((L,), jnp.float32)
    for t in range(RPT // L):
        zz_v[pl.ds(t * L, L)] = jnp.zeros((L,), jnp.float32)
    pltpu.sync_copy(zz_v.at[pl.ds(0, RPT)], od_sp.at[pl.ds(base, RPT)])
    pltpu.sync_copy(zz_v.at[pl.ds(0, RPT)], id_sp.at[pl.ds(base, RPT)])
    plsc.subcore_barrier()

    def dchunk(j, _):
        pltpu.sync_copy(ones_v, od_sp.at[src_v.at[j]], add=True)
        pltpu.sync_copy(ones_v, id_sp.at[dst_v.at[j]], add=True)
        return 0
    lax.fori_loop(0, NCH, dchunk, 0)
    plsc.subcore_barrier()
    pltpu.sync_copy(od_sp.at[pl.ds(base, RPT)], odp.at[cid, pl.ds(base, RPT)])
    pltpu.sync_copy(id_sp.at[pl.ds(base, RPT)], idp.at[cid, pl.ds(base, RPT)])


def _sc_deg(srcw, dstw):
    mesh = plsc.VectorSubcoreMesh(**_MESH)
    f = pl.kernel(
        _sc_deg_body,
        out_type=(jax.ShapeDtypeStruct((NC, N_PAD), jnp.float32),
                  jax.ShapeDtypeStruct((NC, N_PAD), jnp.float32)),
        mesh=mesh,
        compiler_params=pltpu.CompilerParams(needs_layout_passes=False),
        scratch_types=[
            pltpu.VMEM((NCH, CH), jnp.int32),
            pltpu.VMEM((NCH, CH), jnp.int32),
            pltpu.VMEM((CH,), jnp.float32),
            pltpu.VMEM((RPT,), jnp.float32),
            pltpu.VMEM_SHARED((N_PAD,), jnp.float32),
            pltpu.VMEM_SHARED((N_PAD,), jnp.float32),
            pltpu.SemaphoreType.DMA,
        ],
    )
    return f(srcw, dstw)


def _make_sc_agg_body(with_c):
    def body(*refs):
        if with_c:
            (x_hbm, src_hbm, dst_hbm, cw_hbm, ns_hbm, nd_hbm,
             c_out, agg_out,
             src_g, dst_g, c_g, rows_a, rows_b, nd_v, norm_v, agg_sp,
             ga_sem, gb_sem, sa_sem, sb_sem) = refs
        else:
            (x_hbm, src_hbm, dst_hbm, cw_hbm, nd_hbm,
             agg_out,
             src_g, dst_g, c_g, rows_a, rows_b, nd_v, agg_sp,
             ga_sem, gb_sem, sa_sem, sb_sem) = refs
        rows_v = rows_a
        cid = lax.axis_index("c")
        sid = lax.axis_index("s")
        wid = sid * NC + cid
        base = sid * RPT
        pltpu.sync_copy(nd_hbm.at[pl.ds(base, RPT)], nd_v)
        if with_c:
            pltpu.sync_copy(ns_hbm, norm_v)
        _zero_rows(rows_v)
        off = 0
        for sz in SUB:
            pltpu.sync_copy(rows_v.at[pl.ds(0, sz)],
                            agg_sp.at[pl.ds(base + off, sz), :])
            off += sz
        plsc.subcore_barrier()

        def group(grp, _):
            pltpu.sync_copy(src_hbm.at[wid, pl.ds(grp * G, G)], src_g)
            pltpu.sync_copy(dst_hbm.at[wid, pl.ds(grp * G, G)], dst_g)
            pltpu.sync_copy(cw_hbm.at[wid, pl.ds(grp * G, G)], c_g)
            if with_c:
                def cbody(j2, _):
                    for t in range(CH // L):
                        idx = src_g[j2, pl.ds(t * L, L)]
                        nv = plsc.load_gather(norm_v, [idx])
                        c_g[j2, pl.ds(t * L, L)] = (
                            c_g[j2, pl.ds(t * L, L)] * nv)
                    return 0
                lax.fori_loop(0, G, cbody, 0)
                pltpu.sync_copy(c_g, c_out.at[wid, pl.ds(grp * G, G)])
            bufs = ((rows_a, ga_sem, sa_sem), (rows_b, gb_sem, sb_sem))
            pltpu.async_copy(x_hbm.at[src_g.at[0]], rows_a, ga_sem)
            for j in range(G):
                rows, gsem, ssem = bufs[j % 2]
                orows, ogsem, ossem = bufs[(j + 1) % 2]
                pltpu.make_async_copy(x_hbm.at[src_g.at[j]], rows,
                                      gsem).wait()
                if j < G - 1:
                    if j >= 1:
                        pltpu.make_async_copy(
                            orows, agg_sp.at[dst_g.at[j - 1]], ossem).wait()
                    pltpu.async_copy(x_hbm.at[src_g.at[j + 1]], orows, ogsem)

                def scale(t, _, j=j, rows=rows):
                    cvec = c_g[j, pl.ds(t * L, L)]
                    for k in range(L):
                        cv = lax.broadcast(cvec[k], (L,))
                        row = t * L + k
                        for q in range(D // L):
                            rows[row, pl.ds(q * L, L)] = (
                                rows[row, pl.ds(q * L, L)] * cv)
                    return 0
                lax.fori_loop(0, CH // L, scale, 0)
                pltpu.async_copy(rows, agg_sp.at[dst_g.at[j]], ssem, add=True)
            pltpu.make_async_copy(rows_a, agg_sp.at[dst_g.at[G - 2]],
                                  sa_sem).wait()
            pltpu.make_async_copy(rows_b, agg_sp.at[dst_g.at[G - 1]],
                                  sb_sem).wait()
            return 0
        lax.fori_loop(0, NG, group, 0)
        plsc.subcore_barrier()
        off = 0
        for sz in SUB:
            pltpu.sync_copy(agg_sp.at[pl.ds(base + off, sz), :],
                            rows_v.at[pl.ds(0, sz)])

            def nrow(g, _):
                nvec = nd_v[pl.ds(off + g * L, L)]
                for k in range(L):
                    cv = lax.broadcast(nvec[k], (L,))
                    row = g * L + k
                    for q in range(D // L):
                        rows_v[row, pl.ds(q * L, L)] = (
                            rows_v[row, pl.ds(q * L, L)] * cv)
                return 0
            lax.fori_loop(0, sz // L, nrow, 0)
            pltpu.sync_copy(rows_v.at[pl.ds(0, sz)],
                            agg_out.at[cid, pl.ds(base + off, sz), :])
            off += sz
    return body


def _sc_agg(with_c, *args):
    mesh = plsc.VectorSubcoreMesh(**_MESH)
    scratch = [
        pltpu.VMEM((G, CH), jnp.int32),
        pltpu.VMEM((G, CH), jnp.int32),
        pltpu.VMEM((G, CH), jnp.float32),
        pltpu.VMEM((CH, D), jnp.float32),
        pltpu.VMEM((CH, D), jnp.float32),
        pltpu.VMEM((RPT,), jnp.float32),
    ]
    if with_c:
        scratch.append(pltpu.VMEM((N_PAD,), jnp.float32))
    scratch += [
        pltpu.VMEM_SHARED((N_PAD, D), jnp.float32),
        pltpu.SemaphoreType.DMA,
        pltpu.SemaphoreType.DMA,
        pltpu.SemaphoreType.DMA,
        pltpu.SemaphoreType.DMA,
    ]
    out_type = [jax.ShapeDtypeStruct((NC, N_PAD, D), jnp.float32)]
    if with_c:
        out_type = [jax.ShapeDtypeStruct((NW, NCH, CH), jnp.float32)] + out_type
    f = pl.kernel(
        _make_sc_agg_body(with_c),
        out_type=tuple(out_type),
        mesh=mesh,
        compiler_params=pltpu.CompilerParams(needs_layout_passes=False),
        scratch_types=scratch,
    )
    return f(*args)


def _tc_ew_body(e_ref, l1w_ref, l1b_ref, l2w_ref, l2b_ref, o_ref):
    w_eff = jnp.sum(l1w_ref[...] * l2w_ref[...][None, :], axis=1)
    b_eff = jnp.sum(l1b_ref[...] * l2w_ref[...]) + l2b_ref[...][0]
    o_ref[...] = jnp.sum(e_ref[...] * w_eff[None, :], axis=1) + b_eff


def _tc_ew(edges_pad, l1w, l1b, l2w_flat, l2b):
    blk = 2048
    grid = E_PAD // blk
    return pl.pallas_call(
        _tc_ew_body,
        grid=(grid,),
        in_specs=[
            pl.BlockSpec((blk, 16), lambda i: (i, 0)),
            pl.BlockSpec((16, 8), lambda i: (0, 0)),
            pl.BlockSpec((8,), lambda i: (0,)),
            pl.BlockSpec((8,), lambda i: (0,)),
            pl.BlockSpec((1,), lambda i: (0,)),
        ],
        out_specs=pl.BlockSpec((blk,), lambda i: (i,)),
        out_shape=jax.ShapeDtypeStruct((E_PAD,), jnp.float32),
    )(edges_pad, l1w, l1b, l2w_flat, l2b)


def _tc_norm_body(odp_ref, idp_ref, ns_ref, nd_ref):
    od = odp_ref[0, :] + odp_ref[1, :]
    idg = idp_ref[0, :] + idp_ref[1, :]
    ns_ref[...] = lax.rsqrt(jnp.maximum(od, 1.0))
    nd_ref[...] = lax.rsqrt(jnp.maximum(idg, 1.0))


def _tc_norm(odp, idp):
    return pl.pallas_call(
        _tc_norm_body,
        out_shape=(jax.ShapeDtypeStruct((N_PAD,), jnp.float32),
                   jax.ShapeDtypeStruct((N_PAD,), jnp.float32)),
    )(odp, idp)


def _tc_layer_body(aggp_ref, w_ref, b_ref, o_ref):
    a = aggp_ref[0] + aggp_ref[1]
    h = jnp.dot(a, w_ref[...], preferred_element_type=jnp.float32,
                precision=lax.Precision.HIGHEST)
    o_ref[...] = jnp.maximum(h + b_ref[...][None, :], 0.0)


def _tc_layer(aggp, w, b):
    rb = RPT
    grid = N_PAD // rb
    return pl.pallas_call(
        _tc_layer_body,
        grid=(grid,),
        in_specs=[
            pl.BlockSpec((NC, rb, D), lambda i: (0, i, 0)),
            pl.BlockSpec((D, D), lambda i: (0, 0)),
            pl.BlockSpec((D,), lambda i: (0,)),
        ],
        out_specs=pl.BlockSpec((rb, D), lambda i: (i, 0)),
        out_shape=jax.ShapeDtypeStruct((N_PAD, D), jnp.float32),
    )(aggp, w, b)


def kernel(inputs, edge_index, edges, W1, b1, W2, b2, lin1_W, lin1_b,
           lin2_W, lin2_b):
    pad = E_PAD - E
    trash = jnp.full((pad,), N, jnp.int32)
    srcw = jnp.concatenate([edge_index[0], trash]).reshape(NW, NCH, CH)
    dstw = jnp.concatenate([edge_index[1], trash]).reshape(NW, NCH, CH)
    x_pad = jnp.pad(inputs, ((0, N_PAD - N), (0, 0)))
    edges_pad = jnp.pad(edges, ((0, pad), (0, 0)))
    l2w_flat = lin2_W.reshape(8)

    ew = _tc_ew(edges_pad, lin1_W, lin1_b, l2w_flat, lin2_b)
    eww = ew.reshape(NW, NCH, CH)
    odp, idp = _sc_deg(srcw, dstw)
    ns, nd = _tc_norm(odp, idp)
    c_out, agg1 = _sc_agg(True, x_pad, srcw, dstw, eww, ns, nd)
    h = _tc_layer(agg1, W1, b1)
    (agg2,) = _sc_agg(False, h, srcw, dstw, c_out, nd)
    out_full = _tc_layer(agg2, W2, b2)
    return out_full[:N]


# speedup vs baseline: 2.7330x; 1.0596x over previous
import functools

import jax
import jax.numpy as jnp
from jax import lax
from jax.experimental import pallas as pl
from jax.experimental.pallas import tpu as pltpu
from jax.experimental.pallas import tpu_sc as plsc

N = 10000
E = 320000
D = 128
NC = 2
NS = 16
NW = NC * NS
L = 16
CH = 128
NCH = 80
G = 8
NG = NCH // G
EPW = NCH * CH
E_PAD = NW * EPW
N_PAD = 10240
RPT = N_PAD // NS
SUB = (128, 128, 128, 128, 128)

_MESH = dict(core_axis_name="c", subcore_axis_name="s", num_cores=NC,
             num_subcores=NS)


def _zero_rows(rows_v):
    def zr(i, _):
        for q in range(D // L):
            rows_v[i, pl.ds(q * L, L)] = jnp.zeros((L,), jnp.float32)
        return 0
    lax.fori_loop(0, CH, zr, 0)


def _sc_deg_body(src_hbm, dst_hbm, odp, idp, src_v, dst_v, ones_v, zz_v,
                 od_sp, id_sp, sem):
    cid = lax.axis_index("c")
    sid = lax.axis_index("s")
    wid = sid * NC + cid
    base = sid * RPT
    pltpu.sync_copy(src_hbm.at[wid], src_v)
    pltpu.sync_copy(dst_hbm.at[wid], dst_v)
    for t in range(CH // L):
        ones_v[pl.ds(t * L, L)] = jnp.ones((L,), jnp.float32)
    for t in range(RPT // L):
        zz_v[pl.ds(t * L, L)] = jnp.zeros((L,), jnp.float32)
    pltpu.sync_copy(zz_v.at[pl.ds(0, RPT)], od_sp.at[pl.ds(base, RPT)])
    pltpu.sync_copy(zz_v.at[pl.ds(0, RPT)], id_sp.at[pl.ds(base, RPT)])
    plsc.subcore_barrier()

    def dchunk(j, _):
        pltpu.sync_copy(ones_v, od_sp.at[src_v.at[j]], add=True)
        pltpu.sync_copy(ones_v, id_sp.at[dst_v.at[j]], add=True)
        return 0
    lax.fori_loop(0, NCH, dchunk, 0)
    plsc.subcore_barrier()
    pltpu.sync_copy(od_sp.at[pl.ds(base, RPT)], odp.at[cid, pl.ds(base, RPT)])
    pltpu.sync_copy(id_sp.at[pl.ds(base, RPT)], idp.at[cid, pl.ds(base, RPT)])


def _sc_deg(srcw, dstw):
    mesh = plsc.VectorSubcoreMesh(**_MESH)
    f = pl.kernel(
        _sc_deg_body,
        out_type=(jax.ShapeDtypeStruct((NC, N_PAD), jnp.float32),
                  jax.ShapeDtypeStruct((NC, N_PAD), jnp.float32)),
        mesh=mesh,
        compiler_params=pltpu.CompilerParams(needs_layout_passes=False),
        scratch_types=[
            pltpu.VMEM((NCH, CH), jnp.int32),
            pltpu.VMEM((NCH, CH), jnp.int32),
            pltpu.VMEM((CH,), jnp.float32),
            pltpu.VMEM((RPT,), jnp.float32),
            pltpu.VMEM_SHARED((N_PAD,), jnp.float32),
            pltpu.VMEM_SHARED((N_PAD,), jnp.float32),
            pltpu.SemaphoreType.DMA,
        ],
    )
    return f(srcw, dstw)


def _sc_agg_body(x_hbm, src_hbm, dst_hbm, cw_hbm, nd_hbm,
                 agg_out,
                 src_g, dst_g, c_g, rows_a, rows_b, nd_v, agg_sp,
                 ga_sem, gb_sem, sa_sem, sb_sem):
    rows_v = rows_a
    cid = lax.axis_index("c")
    sid = lax.axis_index("s")
    wid = sid * NC + cid
    base = sid * RPT
    pltpu.sync_copy(nd_hbm.at[pl.ds(base, RPT)], nd_v)
    _zero_rows(rows_v)
    off = 0
    for sz in SUB:
        pltpu.sync_copy(rows_v.at[pl.ds(0, sz)],
                        agg_sp.at[pl.ds(base + off, sz), :])
        off += sz
    plsc.subcore_barrier()

    def group(grp, _):
        pltpu.sync_copy(src_hbm.at[wid, pl.ds(grp * G, G)], src_g)
        pltpu.sync_copy(dst_hbm.at[wid, pl.ds(grp * G, G)], dst_g)
        pltpu.sync_copy(cw_hbm.at[wid, pl.ds(grp * G, G)], c_g)
        bufs = ((rows_a, ga_sem, sa_sem), (rows_b, gb_sem, sb_sem))
        pltpu.async_copy(x_hbm.at[src_g.at[0]], rows_a, ga_sem)
        for j in range(G):
            rows, gsem, ssem = bufs[j % 2]
            orows, ogsem, ossem = bufs[(j + 1) % 2]
            pltpu.make_async_copy(x_hbm.at[src_g.at[j]], rows,
                                  gsem).wait()
            if j < G - 1:
                if j >= 1:
                    pltpu.make_async_copy(
                        orows, agg_sp.at[dst_g.at[j - 1]], ossem).wait()
                pltpu.async_copy(x_hbm.at[src_g.at[j + 1]], orows, ogsem)

            def scale(t, _, j=j, rows=rows):
                cvec = c_g[j, pl.ds(t * L, L)]
                for k in range(L):
                    cv = lax.broadcast(cvec[k], (L,))
                    row = t * L + k
                    for q in range(D // L):
                        rows[row, pl.ds(q * L, L)] = (
                            rows[row, pl.ds(q * L, L)] * cv)
                return 0
            lax.fori_loop(0, CH // L, scale, 0)
            pltpu.async_copy(rows, agg_sp.at[dst_g.at[j]], ssem, add=True)
        pltpu.make_async_copy(rows_a, agg_sp.at[dst_g.at[G - 2]],
                              sa_sem).wait()
        pltpu.make_async_copy(rows_b, agg_sp.at[dst_g.at[G - 1]],
                              sb_sem).wait()
        return 0
    lax.fori_loop(0, NG, group, 0)
    plsc.subcore_barrier()
    off = 0
    for sz in SUB:
        pltpu.sync_copy(agg_sp.at[pl.ds(base + off, sz), :],
                        rows_v.at[pl.ds(0, sz)])

        def nrow(g, _, off=off):
            nvec = nd_v[pl.ds(off + g * L, L)]
            for k in range(L):
                cv = lax.broadcast(nvec[k], (L,))
                row = g * L + k
                for q in range(D // L):
                    rows_v[row, pl.ds(q * L, L)] = (
                        rows_v[row, pl.ds(q * L, L)] * cv)
            return 0
        lax.fori_loop(0, sz // L, nrow, 0)
        pltpu.sync_copy(rows_v.at[pl.ds(0, sz)],
                        agg_out.at[cid, pl.ds(base + off, sz), :])
        off += sz


def _sc_agg(x, srcw, dstw, eww, nd):
    mesh = plsc.VectorSubcoreMesh(**_MESH)
    scratch = [
        pltpu.VMEM((G, CH), jnp.int32),
        pltpu.VMEM((G, CH), jnp.int32),
        pltpu.VMEM((G, CH), jnp.float32),
        pltpu.VMEM((CH, D), jnp.float32),
        pltpu.VMEM((CH, D), jnp.float32),
        pltpu.VMEM((RPT,), jnp.float32),
        pltpu.VMEM_SHARED((N_PAD, D), jnp.float32),
        pltpu.SemaphoreType.DMA,
        pltpu.SemaphoreType.DMA,
        pltpu.SemaphoreType.DMA,
        pltpu.SemaphoreType.DMA,
    ]
    f = pl.kernel(
        _sc_agg_body,
        out_type=(jax.ShapeDtypeStruct((NC, N_PAD, D), jnp.float32),),
        mesh=mesh,
        compiler_params=pltpu.CompilerParams(needs_layout_passes=False),
        scratch_types=scratch,
    )
    (agg,) = f(x, srcw, dstw, eww, nd)
    return agg


def _tc_ew_body(e_ref, l1w_ref, l1b_ref, l2w_ref, l2b_ref, o_ref):
    w_eff = jnp.sum(l1w_ref[...] * l2w_ref[...][None, :], axis=1)
    b_eff = jnp.sum(l1b_ref[...] * l2w_ref[...]) + l2b_ref[...][0]
    o_ref[...] = jnp.sum(e_ref[...] * w_eff[None, :], axis=1) + b_eff


def _tc_ew(edges_pad, l1w, l1b, l2w_flat, l2b):
    blk = 2048
    grid = E_PAD // blk
    return pl.pallas_call(
        _tc_ew_body,
        grid=(grid,),
        in_specs=[
            pl.BlockSpec((blk, 16), lambda i: (i, 0)),
            pl.BlockSpec((16, 8), lambda i: (0, 0)),
            pl.BlockSpec((8,), lambda i: (0,)),
            pl.BlockSpec((8,), lambda i: (0,)),
            pl.BlockSpec((1,), lambda i: (0,)),
        ],
        out_specs=pl.BlockSpec((blk,), lambda i: (i,)),
        out_shape=jax.ShapeDtypeStruct((E_PAD,), jnp.float32),
    )(edges_pad, l1w, l1b, l2w_flat, l2b)


def _tc_norm_body(odp_ref, idp_ref, x_ref, ns_ref, nd_ref, xs_ref):
    od = odp_ref[0, :] + odp_ref[1, :]
    idg = idp_ref[0, :] + idp_ref[1, :]
    ns = lax.rsqrt(jnp.maximum(od, 1.0))
    ns_ref[...] = ns
    nd_ref[...] = lax.rsqrt(jnp.maximum(idg, 1.0))
    xs_ref[...] = x_ref[...] * ns[:, None]


def _tc_norm(odp, idp, x_pad):
    return pl.pallas_call(
        _tc_norm_body,
        out_shape=(jax.ShapeDtypeStruct((N_PAD,), jnp.float32),
                   jax.ShapeDtypeStruct((N_PAD,), jnp.float32),
                   jax.ShapeDtypeStruct((N_PAD, D), jnp.float32)),
    )(odp, idp, x_pad)


def _make_tc_layer_body(scale_out):
    def body(aggp_ref, w_ref, b_ref, ns_ref, o_ref):
        a = aggp_ref[0] + aggp_ref[1]
        h = jnp.dot(a, w_ref[...], preferred_element_type=jnp.float32,
                    precision=lax.Precision.HIGHEST)
        h = jnp.maximum(h + b_ref[...][None, :], 0.0)
        if scale_out:
            i = pl.program_id(0)
            nsb = ns_ref[pl.ds(i * RPT, RPT)]
            h = h * nsb[:, None]
        o_ref[...] = h
    return body


def _tc_layer(aggp, w, b, ns, scale_out):
    rb = RPT
    grid = N_PAD // rb
    return pl.pallas_call(
        _make_tc_layer_body(scale_out),
        grid=(grid,),
        in_specs=[
            pl.BlockSpec((NC, rb, D), lambda i: (0, i, 0)),
            pl.BlockSpec((D, D), lambda i: (0, 0)),
            pl.BlockSpec((D,), lambda i: (0,)),
            pl.BlockSpec((N_PAD,), lambda i: (0,)),
        ],
        out_specs=pl.BlockSpec((rb, D), lambda i: (i, 0)),
        out_shape=jax.ShapeDtypeStruct((N_PAD, D), jnp.float32),
    )(aggp, w, b, ns)


def kernel(inputs, edge_index, edges, W1, b1, W2, b2, lin1_W, lin1_b,
           lin2_W, lin2_b):
    pad = E_PAD - E
    trash = jnp.full((pad,), N, jnp.int32)
    srcw = jnp.concatenate([edge_index[0], trash]).reshape(NW, NCH, CH)
    dstw = jnp.concatenate([edge_index[1], trash]).reshape(NW, NCH, CH)
    x_pad = jnp.pad(inputs, ((0, N_PAD - N), (0, 0)))
    edges_pad = jnp.pad(edges, ((0, pad), (0, 0)))
    l2w_flat = lin2_W.reshape(8)

    ew = _tc_ew(edges_pad, lin1_W, lin1_b, l2w_flat, lin2_b)
    eww = ew.reshape(NW, NCH, CH)
    odp, idp = _sc_deg(srcw, dstw)
    ns, nd, x_s = _tc_norm(odp, idp, x_pad)
    agg1 = _sc_agg(x_s, srcw, dstw, eww, nd)
    h_s = _tc_layer(agg1, W1, b1, ns, True)
    agg2 = _sc_agg(h_s, srcw, dstw, eww, nd)
    out_full = _tc_layer(agg2, W2, b2, ns, False)
    return out_full[:N]
